# Initial kernel scaffold; baseline (speedup 1.0000x reference)
import jax
import jax.numpy as jnp
from jax.experimental import pallas as pl


def kernel(x, edge_index, W1, b1, gamma1, beta1, W2, b2, gamma2, beta2):
    raise NotImplementedError("write your pallas kernel here")


# speedup vs baseline: 23.2419x; 23.2419x over previous
import functools

import jax
import jax.numpy as jnp
from jax import lax
from jax.experimental import pallas as pl
from jax.experimental.pallas import tpu as pltpu
from jax.experimental.pallas import tpu_sc as plsc

N = 10000
D = 128
E = 320000
EPS = 1e-5

NC = 2
NS = 16
NW = NC * NS
K = 128
EW = E // NW
NB = EW // K + 1
EWP = NB * K
NPAD = 10112
RPT = NPAD // NS

_mesh = plsc.VectorSubcoreMesh(core_axis_name="c", subcore_axis_name="s")


def _deg_body(dst_hbm, out_hbm, dst_v, hist_v):
    c = lax.axis_index("c")
    s = lax.axis_index("s")
    wid = s * NC + c
    pltpu.sync_copy(dst_hbm.at[wid], dst_v)

    def _zero(i, carry):
        hist_v[pl.ds(i * 16, 16)] = jnp.zeros((16,), jnp.float32)
        return carry

    lax.fori_loop(0, NPAD // 16, _zero, 0, unroll=4)

    ones = jnp.ones((16,), jnp.float32)

    def _hist(i, carry):
        j = i // 8
        t = i % 8
        idx = dst_v[j, pl.ds(t * 16, 16)]
        plsc.addupdate_scatter(hist_v, [idx], ones)
        return carry

    lax.fori_loop(0, NB * 8, _hist, 0, unroll=4)
    pltpu.sync_copy(hist_v, out_hbm.at[wid])


_deg_hist = functools.partial(
    pl.kernel,
    out_type=jax.ShapeDtypeStruct((NW, NPAD), jnp.float32),
    mesh=_mesh,
    scratch_types=[
        pltpu.VMEM((NB, K), jnp.int32),
        pltpu.VMEM((NPAD,), jnp.float32),
    ],
    compiler_params=pltpu.CompilerParams(needs_layout_passes=False),
)(_deg_body)


def _agg_body(g_hbm, src_hbm, dst_hbm, z_hbm, out_hbm, src_v, dst_v, rows_v,
              acc_sh, sem):
    c = lax.axis_index("c")
    s = lax.axis_index("s")
    wid = s * NC + c
    pltpu.sync_copy(src_hbm.at[wid], src_v)
    pltpu.sync_copy(dst_hbm.at[wid], dst_v)
    pltpu.sync_copy(z_hbm.at[pl.ds(s * RPT, RPT)],
                    acc_sh.at[pl.ds(s * RPT, RPT)])
    plsc.subcore_barrier()

    def _edge(j, carry):
        pltpu.async_copy(g_hbm.at[src_v.at[j]], rows_v, sem).wait()
        pltpu.sync_copy(rows_v, acc_sh.at[dst_v.at[j]], add=True)
        return carry

    lax.fori_loop(0, NB, _edge, 0)
    plsc.subcore_barrier()
    pltpu.sync_copy(acc_sh.at[pl.ds(s * RPT, RPT)],
                    out_hbm.at[c, pl.ds(s * RPT, RPT)])


_edge_agg = functools.partial(
    pl.kernel,
    out_type=jax.ShapeDtypeStruct((NC, NPAD, D), jnp.float32),
    mesh=_mesh,
    scratch_types=[
        pltpu.VMEM((NB, K), jnp.int32),
        pltpu.VMEM((NB, K), jnp.int32),
        pltpu.VMEM((K, D), jnp.float32),
        pltpu.VMEM_SHARED((NPAD, D), jnp.float32),
        pltpu.SemaphoreType.DMA,
    ],
)(_agg_body)


def _tc1_body(hist_ref, x_ref, w_ref, g_ref, dis_ref):
    deg = jnp.sum(hist_ref[...], axis=0) + 1.0
    dis = lax.rsqrt(deg)[:, None]
    h = jnp.dot(x_ref[...], w_ref[...], preferred_element_type=jnp.float32)
    g_ref[...] = h * dis[:N]
    dis_ref[...] = dis


_tc1 = pl.pallas_call(
    _tc1_body,
    out_shape=[
        jax.ShapeDtypeStruct((N, D), jnp.float32),
        jax.ShapeDtypeStruct((NPAD, 1), jnp.float32),
    ],
)


def _tc2_body(acc_ref, g_ref, dis_ref, w_ref, gamma_ref, beta_ref, g2_ref):
    dis = dis_ref[...][:N]
    pre = (acc_ref[0, :N, :] + acc_ref[1, :N, :] + g_ref[...]) * dis
    mu = jnp.mean(pre, axis=0)
    var = jnp.mean((pre - mu) ** 2, axis=0)
    y = (pre - mu) * lax.rsqrt(var + EPS) * gamma_ref[...] + beta_ref[...]
    y = jnp.maximum(y, 0.0)
    g2_ref[...] = jnp.dot(y, w_ref[...],
                          preferred_element_type=jnp.float32) * dis


_tc2 = pl.pallas_call(
    _tc2_body,
    out_shape=jax.ShapeDtypeStruct((N, D), jnp.float32),
)


def _tc3_body(acc_ref, g_ref, dis_ref, gamma_ref, beta_ref, out_ref):
    dis = dis_ref[...][:N]
    pre = (acc_ref[0, :N, :] + acc_ref[1, :N, :] + g_ref[...]) * dis
    mu = jnp.mean(pre, axis=0)
    var = jnp.mean((pre - mu) ** 2, axis=0)
    out_ref[...] = (pre - mu) * lax.rsqrt(var + EPS) * gamma_ref[...] \
        + beta_ref[...]


_tc3 = pl.pallas_call(
    _tc3_body,
    out_shape=jax.ShapeDtypeStruct((N, D), jnp.float32),
)


def kernel(x, edge_index, W1, b1, gamma1, beta1, W2, b2, gamma2, beta2):
    src = edge_index[0]
    dst = edge_index[1]
    npad = EWP - EW
    pad_lane = (jnp.arange(npad, dtype=jnp.int32) % 32)
    src_p = jnp.concatenate(
        [src.reshape(NW, EW), jnp.broadcast_to(pad_lane, (NW, npad))],
        axis=1).reshape(NW, NB, K)
    dst_p = jnp.concatenate(
        [dst.reshape(NW, EW), jnp.broadcast_to(N + pad_lane, (NW, npad))],
        axis=1).reshape(NW, NB, K)
    zeros = jnp.zeros((NPAD, D), jnp.float32)

    hist = _deg_hist(dst_p)
    g1, dis = _tc1(hist, x, W1)
    acc1 = _edge_agg(g1, src_p, dst_p, zeros)
    g2 = _tc2(acc1, g1, dis, W2, gamma1, beta1)
    acc2 = _edge_agg(g2, src_p, dst_p, zeros)
    out = _tc3(acc2, g2, dis, gamma2, beta2)
    return out


# speedup vs baseline: 28.6428x; 1.2324x over previous
import functools

import jax
import jax.numpy as jnp
from jax import lax
from jax.experimental import pallas as pl
from jax.experimental.pallas import tpu as pltpu
from jax.experimental.pallas import tpu_sc as plsc

N = 10000
D = 128
E = 320000
EPS = 1e-5

NC = 2
NS = 16
NW = NC * NS
K = 128
EW = E // NW
NB = 80
EWP = NB * K
BLK = 16
NBLK = NB // BLK
NPAD = 10112
RPT = NPAD // NS

_mesh = plsc.VectorSubcoreMesh(core_axis_name="c", subcore_axis_name="s")


def _deg_body(dst_hbm, out_hbm, dst_v, hist_v):
    c = lax.axis_index("c")
    s = lax.axis_index("s")
    wid = s * NC + c
    pltpu.sync_copy(dst_hbm.at[wid], dst_v)

    def _zero(i, carry):
        hist_v[pl.ds(i * 16, 16)] = jnp.zeros((16,), jnp.float32)
        return carry

    lax.fori_loop(0, NPAD // 16, _zero, 0, unroll=4)

    ones = jnp.ones((16,), jnp.float32)

    def _hist(i, carry):
        j = i // (K // 16)
        t = i % (K // 16)
        idx = dst_v[j, pl.ds(t * 16, 16)]
        plsc.addupdate_scatter(hist_v, [idx], ones)
        return carry

    lax.fori_loop(0, NB * (K // 16), _hist, 0, unroll=4)
    pltpu.sync_copy(hist_v, out_hbm.at[wid])


_deg_hist = functools.partial(
    pl.kernel,
    out_type=jax.ShapeDtypeStruct((NW, NPAD), jnp.float32),
    mesh=_mesh,
    scratch_types=[
        pltpu.VMEM((NB, K), jnp.int32),
        pltpu.VMEM((NPAD,), jnp.float32),
    ],
    compiler_params=pltpu.CompilerParams(needs_layout_passes=False),
)(_deg_body)


def _agg_body(g_hbm, src_hbm, dst_hbm, z_hbm, out_hbm, src_v, dst_v, rows_v,
              acc_sh, gsem, isem):
    c = lax.axis_index("c")
    s = lax.axis_index("s")
    wid = s * NC + c
    pltpu.sync_copy(z_hbm.at[pl.ds(s * RPT, RPT)],
                    acc_sh.at[pl.ds(s * RPT, RPT)])
    pltpu.sync_copy(src_hbm.at[wid, pl.ds(0, BLK)], src_v.at[0])
    pltpu.sync_copy(dst_hbm.at[wid, pl.ds(0, BLK)], dst_v.at[0])
    plsc.subcore_barrier()

    pltpu.async_copy(g_hbm.at[src_v.at[0, 0]], rows_v.at[0], gsem.at[0])

    def _blk(b, carry):
        pb = lax.rem(b, 2)

        @pl.when(b + 1 < NBLK)
        def _prefetch_idx():
            pltpu.async_copy(src_hbm.at[wid, pl.ds((b + 1) * BLK, BLK)],
                             src_v.at[1 - pb], isem)
            pltpu.async_copy(dst_hbm.at[wid, pl.ds((b + 1) * BLK, BLK)],
                             dst_v.at[1 - pb], isem)

        def _edge(t, carry2):
            j = b * BLK + t
            p = lax.rem(j, 2)
            pltpu.make_async_copy(g_hbm.at[src_v.at[pb, t]], rows_v.at[p],
                                  gsem.at[p]).wait()

            @pl.when(t + 1 < BLK)
            def _prefetch_same_blk():
                pltpu.async_copy(g_hbm.at[src_v.at[pb, t + 1]],
                                 rows_v.at[1 - p], gsem.at[1 - p])

            @pl.when(jnp.logical_and(t + 1 >= BLK, b + 1 < NBLK))
            def _prefetch_next_blk():
                pltpu.make_async_copy(
                    src_hbm.at[wid, pl.ds((b + 1) * BLK, BLK)],
                    src_v.at[1 - pb], isem).wait()
                pltpu.make_async_copy(
                    dst_hbm.at[wid, pl.ds((b + 1) * BLK, BLK)],
                    dst_v.at[1 - pb], isem).wait()
                pltpu.async_copy(g_hbm.at[src_v.at[1 - pb, 0]],
                                 rows_v.at[1 - p], gsem.at[1 - p])

            pltpu.sync_copy(rows_v.at[p], acc_sh.at[dst_v.at[pb, t]],
                            add=True)
            return carry2

        lax.fori_loop(0, BLK, _edge, carry)
        return carry

    lax.fori_loop(0, NBLK, _blk, 0)
    plsc.subcore_barrier()
    pltpu.sync_copy(acc_sh.at[pl.ds(s * RPT, RPT)],
                    out_hbm.at[c, pl.ds(s * RPT, RPT)])


_edge_agg = functools.partial(
    pl.kernel,
    out_type=jax.ShapeDtypeStruct((NC, NPAD, D), jnp.float32),
    mesh=_mesh,
    scratch_types=[
        pltpu.VMEM((2, BLK, K), jnp.int32),
        pltpu.VMEM((2, BLK, K), jnp.int32),
        pltpu.VMEM((2, K, D), jnp.float32),
        pltpu.VMEM_SHARED((NPAD, D), jnp.float32),
        pltpu.SemaphoreType.DMA((2,)),
        pltpu.SemaphoreType.DMA,
    ],
)(_agg_body)


def _tc1_body(hist_ref, x_ref, w_ref, g_ref, dis_ref):
    deg = jnp.sum(hist_ref[...], axis=0) + 1.0
    dis = lax.rsqrt(deg)[:, None]
    h = jnp.dot(x_ref[...], w_ref[...], preferred_element_type=jnp.float32)
    g_ref[...] = h * dis[:N]
    dis_ref[...] = dis


_tc1 = pl.pallas_call(
    _tc1_body,
    out_shape=[
        jax.ShapeDtypeStruct((N, D), jnp.float32),
        jax.ShapeDtypeStruct((NPAD, 1), jnp.float32),
    ],
)


def _tc2_body(acc_ref, g_ref, dis_ref, w_ref, gamma_ref, beta_ref, g2_ref):
    dis = dis_ref[...][:N]
    pre = (acc_ref[0, :N, :] + acc_ref[1, :N, :] + g_ref[...]) * dis
    mu = jnp.mean(pre, axis=0)
    var = jnp.mean((pre - mu) ** 2, axis=0)
    y = (pre - mu) * lax.rsqrt(var + EPS) * gamma_ref[...] + beta_ref[...]
    y = jnp.maximum(y, 0.0)
    g2_ref[...] = jnp.dot(y, w_ref[...],
                          preferred_element_type=jnp.float32) * dis


_tc2 = pl.pallas_call(
    _tc2_body,
    out_shape=jax.ShapeDtypeStruct((N, D), jnp.float32),
)


def _tc3_body(acc_ref, g_ref, dis_ref, gamma_ref, beta_ref, out_ref):
    dis = dis_ref[...][:N]
    pre = (acc_ref[0, :N, :] + acc_ref[1, :N, :] + g_ref[...]) * dis
    mu = jnp.mean(pre, axis=0)
    var = jnp.mean((pre - mu) ** 2, axis=0)
    out_ref[...] = (pre - mu) * lax.rsqrt(var + EPS) * gamma_ref[...] \
        + beta_ref[...]


_tc3 = pl.pallas_call(
    _tc3_body,
    out_shape=jax.ShapeDtypeStruct((N, D), jnp.float32),
)


def kernel(x, edge_index, W1, b1, gamma1, beta1, W2, b2, gamma2, beta2):
    src = edge_index[0]
    dst = edge_index[1]
    npad = EWP - EW
    pad_lane = (jnp.arange(npad, dtype=jnp.int32) % 32)
    src_p = jnp.concatenate(
        [src.reshape(NW, EW), jnp.broadcast_to(pad_lane, (NW, npad))],
        axis=1).reshape(NW, NB, K)
    dst_p = jnp.concatenate(
        [dst.reshape(NW, EW), jnp.broadcast_to(N + pad_lane, (NW, npad))],
        axis=1).reshape(NW, NB, K)
    zeros = jnp.zeros((NPAD, D), jnp.float32)

    hist = _deg_hist(dst_p)
    g1, dis = _tc1(hist, x, W1)
    acc1 = _edge_agg(g1, src_p, dst_p, zeros)
    g2 = _tc2(acc1, g1, dis, W2, gamma1, beta1)
    acc2 = _edge_agg(g2, src_p, dst_p, zeros)
    out = _tc3(acc2, g2, dis, gamma2, beta2)
    return out


# speedup vs baseline: 29.4444x; 1.0280x over previous
import functools

import jax
import jax.numpy as jnp
from jax import lax
from jax.experimental import pallas as pl
from jax.experimental.pallas import tpu as pltpu
from jax.experimental.pallas import tpu_sc as plsc

N = 10000
D = 128
E = 320000
EPS = 1e-5

NC = 2
NS = 16
NW = NC * NS
K = 128
EW = E // NW
NB = 80
EWP = NB * K
BLK = 16
NBLK = NB // BLK
NPAD = 10112
RPT = NPAD // NS

_mesh = plsc.VectorSubcoreMesh(core_axis_name="c", subcore_axis_name="s")


def _deg_body(dst_hbm, out_hbm, dst_v, hist_v):
    c = lax.axis_index("c")
    s = lax.axis_index("s")
    wid = s * NC + c
    pltpu.sync_copy(dst_hbm.at[wid], dst_v)

    def _zero(i, carry):
        hist_v[pl.ds(i * 16, 16)] = jnp.zeros((16,), jnp.float32)
        return carry

    lax.fori_loop(0, NPAD // 16, _zero, 0, unroll=4)

    ones = jnp.ones((16,), jnp.float32)

    def _hist(i, carry):
        j = i // (K // 16)
        t = i % (K // 16)
        idx = dst_v[j, pl.ds(t * 16, 16)]
        plsc.addupdate_scatter(hist_v, [idx], ones)
        return carry

    lax.fori_loop(0, NB * (K // 16), _hist, 0, unroll=4)
    pltpu.sync_copy(hist_v, out_hbm.at[wid])


_deg_hist = functools.partial(
    pl.kernel,
    out_type=jax.ShapeDtypeStruct((NW, NPAD), jnp.float32),
    mesh=_mesh,
    scratch_types=[
        pltpu.VMEM((NB, K), jnp.int32),
        pltpu.VMEM((NPAD,), jnp.float32),
    ],
    compiler_params=pltpu.CompilerParams(needs_layout_passes=False),
)(_deg_body)


def _agg_body(g_hbm, src_hbm, dst_hbm, z_hbm, out_hbm, src_v, dst_v, rows_v,
              acc_sh, gsem, ssem, isem):
    c = lax.axis_index("c")
    s = lax.axis_index("s")
    wid = s * NC + c
    pltpu.sync_copy(z_hbm.at[pl.ds(s * RPT, RPT)],
                    acc_sh.at[pl.ds(s * RPT, RPT)])
    pltpu.sync_copy(src_hbm.at[wid, pl.ds(0, BLK)], src_v.at[0])
    pltpu.sync_copy(dst_hbm.at[wid, pl.ds(0, BLK)], dst_v.at[0])
    plsc.subcore_barrier()

    pltpu.async_copy(g_hbm.at[src_v.at[0, 0]], rows_v.at[0], gsem.at[0])

    def _blk(b, carry):
        pb = lax.rem(b, 2)

        @pl.when(b + 1 < NBLK)
        def _prefetch_idx():
            pltpu.async_copy(src_hbm.at[wid, pl.ds((b + 1) * BLK, BLK)],
                             src_v.at[1 - pb], isem)
            pltpu.async_copy(dst_hbm.at[wid, pl.ds((b + 1) * BLK, BLK)],
                             dst_v.at[1 - pb], isem)

        def _edge(t, carry2):
            j = b * BLK + t
            p = lax.rem(j, 2)
            pltpu.make_async_copy(g_hbm.at[src_v.at[pb, t]], rows_v.at[p],
                                  gsem.at[p]).wait()
            pltpu.async_copy(rows_v.at[p], acc_sh.at[dst_v.at[pb, t]],
                             ssem.at[p], add=True)

            @pl.when(j >= 1)
            def _drain_prev_scatter():
                pltpu.make_async_copy(rows_v.at[1 - p],
                                      acc_sh.at[dst_v.at[pb, t]],
                                      ssem.at[1 - p]).wait()

            @pl.when(t + 1 < BLK)
            def _prefetch_same_blk():
                pltpu.async_copy(g_hbm.at[src_v.at[pb, t + 1]],
                                 rows_v.at[1 - p], gsem.at[1 - p])

            @pl.when(jnp.logical_and(t + 1 >= BLK, b + 1 < NBLK))
            def _prefetch_next_blk():
                pltpu.make_async_copy(
                    src_hbm.at[wid, pl.ds((b + 1) * BLK, BLK)],
                    src_v.at[1 - pb], isem).wait()
                pltpu.make_async_copy(
                    dst_hbm.at[wid, pl.ds((b + 1) * BLK, BLK)],
                    dst_v.at[1 - pb], isem).wait()
                pltpu.async_copy(g_hbm.at[src_v.at[1 - pb, 0]],
                                 rows_v.at[1 - p], gsem.at[1 - p])

            return carry2

        lax.fori_loop(0, BLK, _edge, carry)
        return carry

    lax.fori_loop(0, NBLK, _blk, 0)
    pltpu.make_async_copy(rows_v.at[(NB - 1) % 2],
                          acc_sh.at[dst_v.at[(NBLK - 1) % 2, BLK - 1]],
                          ssem.at[(NB - 1) % 2]).wait()
    plsc.subcore_barrier()
    pltpu.sync_copy(acc_sh.at[pl.ds(s * RPT, RPT)],
                    out_hbm.at[c, pl.ds(s * RPT, RPT)])


_edge_agg = functools.partial(
    pl.kernel,
    out_type=jax.ShapeDtypeStruct((NC, NPAD, D), jnp.float32),
    mesh=_mesh,
    scratch_types=[
        pltpu.VMEM((2, BLK, K), jnp.int32),
        pltpu.VMEM((2, BLK, K), jnp.int32),
        pltpu.VMEM((2, K, D), jnp.float32),
        pltpu.VMEM_SHARED((NPAD, D), jnp.float32),
        pltpu.SemaphoreType.DMA((2,)),
        pltpu.SemaphoreType.DMA((2,)),
        pltpu.SemaphoreType.DMA,
    ],
)(_agg_body)


def _tc1_body(hist_ref, x_ref, w_ref, g_ref, dis_ref):
    deg = jnp.sum(hist_ref[...], axis=0) + 1.0
    dis = lax.rsqrt(deg)[:, None]
    h = jnp.dot(x_ref[...], w_ref[...], preferred_element_type=jnp.float32)
    g_ref[...] = h * dis[:N]
    dis_ref[...] = dis


_tc1 = pl.pallas_call(
    _tc1_body,
    out_shape=[
        jax.ShapeDtypeStruct((N, D), jnp.float32),
        jax.ShapeDtypeStruct((NPAD, 1), jnp.float32),
    ],
)


def _tc2_body(acc_ref, g_ref, dis_ref, w_ref, gamma_ref, beta_ref, g2_ref):
    dis = dis_ref[...][:N]
    pre = (acc_ref[0, :N, :] + acc_ref[1, :N, :] + g_ref[...]) * dis
    mu = jnp.mean(pre, axis=0)
    var = jnp.mean((pre - mu) ** 2, axis=0)
    y = (pre - mu) * lax.rsqrt(var + EPS) * gamma_ref[...] + beta_ref[...]
    y = jnp.maximum(y, 0.0)
    g2_ref[...] = jnp.dot(y, w_ref[...],
                          preferred_element_type=jnp.float32) * dis


_tc2 = pl.pallas_call(
    _tc2_body,
    out_shape=jax.ShapeDtypeStruct((N, D), jnp.float32),
)


def _tc3_body(acc_ref, g_ref, dis_ref, gamma_ref, beta_ref, out_ref):
    dis = dis_ref[...][:N]
    pre = (acc_ref[0, :N, :] + acc_ref[1, :N, :] + g_ref[...]) * dis
    mu = jnp.mean(pre, axis=0)
    var = jnp.mean((pre - mu) ** 2, axis=0)
    out_ref[...] = (pre - mu) * lax.rsqrt(var + EPS) * gamma_ref[...] \
        + beta_ref[...]


_tc3 = pl.pallas_call(
    _tc3_body,
    out_shape=jax.ShapeDtypeStruct((N, D), jnp.float32),
)


def kernel(x, edge_index, W1, b1, gamma1, beta1, W2, b2, gamma2, beta2):
    src = edge_index[0]
    dst = edge_index[1]
    npad = EWP - EW
    pad_lane = (jnp.arange(npad, dtype=jnp.int32) % (NPAD - N))
    src_p = jnp.concatenate(
        [src.reshape(NW, EW), jnp.broadcast_to(pad_lane, (NW, npad))],
        axis=1).reshape(NW, NB, K)
    dst_p = jnp.concatenate(
        [dst.reshape(NW, EW), jnp.broadcast_to(N + pad_lane, (NW, npad))],
        axis=1).reshape(NW, NB, K)
    zeros = jnp.zeros((NPAD, D), jnp.float32)

    hist = _deg_hist(dst_p)
    g1, dis = _tc1(hist, x, W1)
    acc1 = _edge_agg(g1, src_p, dst_p, zeros)
    g2 = _tc2(acc1, g1, dis, W2, gamma1, beta1)
    acc2 = _edge_agg(g2, src_p, dst_p, zeros)
    out = _tc3(acc2, g2, dis, gamma2, beta2)
    return out


# speedup vs baseline: 33.8002x; 1.1479x over previous
import functools

import jax
import jax.numpy as jnp
from jax import lax
from jax.experimental import pallas as pl
from jax.experimental.pallas import tpu as pltpu
from jax.experimental.pallas import tpu_sc as plsc

N = 10000
D = 128
E = 320000
EPS = 1e-5

NC = 2
NS = 16
NW = NC * NS
K = 128
EW = E // NW
NB = 80
EWP = NB * K
BLK = 16
NBLK = NB // BLK
NPAD = 10112
RPT = NPAD // NS

_mesh = plsc.VectorSubcoreMesh(core_axis_name="c", subcore_axis_name="s")


def _deg_body(dst_hbm, out_hbm, dst_v, hist_v):
    c = lax.axis_index("c")
    s = lax.axis_index("s")
    wid = s * NC + c
    pltpu.sync_copy(dst_hbm.at[wid], dst_v)

    def _zero(i, carry):
        hist_v[pl.ds(i * 16, 16)] = jnp.zeros((16,), jnp.float32)
        return carry

    lax.fori_loop(0, NPAD // 16, _zero, 0, unroll=4)

    ones = jnp.ones((16,), jnp.float32)

    def _hist(i, carry):
        j = i // (K // 16)
        t = i % (K // 16)
        idx = dst_v[j, pl.ds(t * 16, 16)]
        plsc.addupdate_scatter(hist_v, [idx], ones)
        return carry

    lax.fori_loop(0, NB * (K // 16), _hist, 0, unroll=4)
    pltpu.sync_copy(hist_v, out_hbm.at[wid])


_deg_hist = functools.partial(
    pl.kernel,
    out_type=jax.ShapeDtypeStruct((NW, NPAD), jnp.float32),
    mesh=_mesh,
    scratch_types=[
        pltpu.VMEM((NB, K), jnp.int32),
        pltpu.VMEM((NPAD,), jnp.float32),
    ],
    compiler_params=pltpu.CompilerParams(needs_layout_passes=False),
)(_deg_body)


def _agg_body(g_hbm, src_hbm, dst_hbm, z_hbm, out_hbm, src_v, dst_v, rows_v,
              acc_sh, gsem, ssem, isem):
    c = lax.axis_index("c")
    s = lax.axis_index("s")
    wid = s * NC + c
    pltpu.sync_copy(z_hbm.at[pl.ds(s * RPT, RPT)],
                    acc_sh.at[pl.ds(s * RPT, RPT)])
    pltpu.sync_copy(src_hbm.at[wid, pl.ds(0, BLK)], src_v.at[0])
    pltpu.sync_copy(dst_hbm.at[wid, pl.ds(0, BLK)], dst_v.at[0])
    plsc.subcore_barrier()

    pltpu.async_copy(g_hbm.at[src_v.at[0, 0]], rows_v.at[0], gsem.at[0])

    def _blk(b, carry):
        pb = lax.rem(b, 2)

        @pl.when(b + 1 < NBLK)
        def _prefetch_idx():
            pltpu.async_copy(src_hbm.at[wid, pl.ds((b + 1) * BLK, BLK)],
                             src_v.at[1 - pb], isem)
            pltpu.async_copy(dst_hbm.at[wid, pl.ds((b + 1) * BLK, BLK)],
                             dst_v.at[1 - pb], isem)

        def _edge(t, carry2):
            j = b * BLK + t
            p = lax.rem(j, 2)

            @pl.when(j >= 1)
            def _drain_prev_scatter():
                pltpu.make_async_copy(rows_v.at[1 - p],
                                      acc_sh.at[dst_v.at[pb, t]],
                                      ssem.at[1 - p]).wait()

            @pl.when(t + 1 < BLK)
            def _prefetch_same_blk():
                pltpu.async_copy(g_hbm.at[src_v.at[pb, t + 1]],
                                 rows_v.at[1 - p], gsem.at[1 - p])

            @pl.when(jnp.logical_and(t + 1 >= BLK, b + 1 < NBLK))
            def _prefetch_next_blk():
                pltpu.make_async_copy(
                    src_hbm.at[wid, pl.ds((b + 1) * BLK, BLK)],
                    src_v.at[1 - pb], isem).wait()
                pltpu.make_async_copy(
                    dst_hbm.at[wid, pl.ds((b + 1) * BLK, BLK)],
                    dst_v.at[1 - pb], isem).wait()
                pltpu.async_copy(g_hbm.at[src_v.at[1 - pb, 0]],
                                 rows_v.at[1 - p], gsem.at[1 - p])

            pltpu.make_async_copy(g_hbm.at[src_v.at[pb, t]], rows_v.at[p],
                                  gsem.at[p]).wait()
            pltpu.async_copy(rows_v.at[p], acc_sh.at[dst_v.at[pb, t]],
                             ssem.at[p], add=True)

            return carry2

        lax.fori_loop(0, BLK, _edge, carry)
        return carry

    lax.fori_loop(0, NBLK, _blk, 0)
    pltpu.make_async_copy(rows_v.at[(NB - 1) % 2],
                          acc_sh.at[dst_v.at[(NBLK - 1) % 2, BLK - 1]],
                          ssem.at[(NB - 1) % 2]).wait()
    plsc.subcore_barrier()
    pltpu.sync_copy(acc_sh.at[pl.ds(s * RPT, RPT)],
                    out_hbm.at[c, pl.ds(s * RPT, RPT)])


_edge_agg = functools.partial(
    pl.kernel,
    out_type=jax.ShapeDtypeStruct((NC, NPAD, D), jnp.float32),
    mesh=_mesh,
    scratch_types=[
        pltpu.VMEM((2, BLK, K), jnp.int32),
        pltpu.VMEM((2, BLK, K), jnp.int32),
        pltpu.VMEM((2, K, D), jnp.float32),
        pltpu.VMEM_SHARED((NPAD, D), jnp.float32),
        pltpu.SemaphoreType.DMA((2,)),
        pltpu.SemaphoreType.DMA((2,)),
        pltpu.SemaphoreType.DMA,
    ],
)(_agg_body)


def _tc1_body(hist_ref, x_ref, w_ref, g_ref, dis_ref):
    deg = jnp.sum(hist_ref[...], axis=0) + 1.0
    dis = lax.rsqrt(deg)[:, None]
    h = jnp.dot(x_ref[...], w_ref[...], preferred_element_type=jnp.float32)
    g_ref[...] = h * dis[:N]
    dis_ref[...] = dis


_tc1 = pl.pallas_call(
    _tc1_body,
    out_shape=[
        jax.ShapeDtypeStruct((N, D), jnp.float32),
        jax.ShapeDtypeStruct((NPAD, 1), jnp.float32),
    ],
)


def _tc2_body(acc_ref, g_ref, dis_ref, w_ref, gamma_ref, beta_ref, g2_ref):
    dis = dis_ref[...][:N]
    pre = (acc_ref[0, :N, :] + acc_ref[1, :N, :] + g_ref[...]) * dis
    mu = jnp.mean(pre, axis=0)
    var = jnp.mean((pre - mu) ** 2, axis=0)
    y = (pre - mu) * lax.rsqrt(var + EPS) * gamma_ref[...] + beta_ref[...]
    y = jnp.maximum(y, 0.0)
    g2_ref[...] = jnp.dot(y, w_ref[...],
                          preferred_element_type=jnp.float32) * dis


_tc2 = pl.pallas_call(
    _tc2_body,
    out_shape=jax.ShapeDtypeStruct((N, D), jnp.float32),
)


def _tc3_body(acc_ref, g_ref, dis_ref, gamma_ref, beta_ref, out_ref):
    dis = dis_ref[...][:N]
    pre = (acc_ref[0, :N, :] + acc_ref[1, :N, :] + g_ref[...]) * dis
    mu = jnp.mean(pre, axis=0)
    var = jnp.mean((pre - mu) ** 2, axis=0)
    out_ref[...] = (pre - mu) * lax.rsqrt(var + EPS) * gamma_ref[...] \
        + beta_ref[...]


_tc3 = pl.pallas_call(
    _tc3_body,
    out_shape=jax.ShapeDtypeStruct((N, D), jnp.float32),
)


def kernel(x, edge_index, W1, b1, gamma1, beta1, W2, b2, gamma2, beta2):
    src = edge_index[0]
    dst = edge_index[1]
    npad = EWP - EW
    pad_lane = (jnp.arange(npad, dtype=jnp.int32) % (NPAD - N))
    src_p = jnp.concatenate(
        [src.reshape(NW, EW), jnp.broadcast_to(pad_lane, (NW, npad))],
        axis=1).reshape(NW, NB, K)
    dst_p = jnp.concatenate(
        [dst.reshape(NW, EW), jnp.broadcast_to(N + pad_lane, (NW, npad))],
        axis=1).reshape(NW, NB, K)
    zeros = jnp.zeros((NPAD, D), jnp.float32)

    hist = _deg_hist(dst_p)
    g1, dis = _tc1(hist, x, W1)
    acc1 = _edge_agg(g1, src_p, dst_p, zeros)
    g2 = _tc2(acc1, g1, dis, W2, gamma1, beta1)
    acc2 = _edge_agg(g2, src_p, dst_p, zeros)
    out = _tc3(acc2, g2, dis, gamma2, beta2)
    return out


# speedup vs baseline: 36.5772x; 1.0822x over previous
import functools

import jax
import jax.numpy as jnp
from jax import lax
from jax.experimental import pallas as pl
from jax.experimental.pallas import tpu as pltpu
from jax.experimental.pallas import tpu_sc as plsc

N = 10000
D = 128
E = 320000
EPS = 1e-5

NC = 2
NS = 16
NW = NC * NS
K = 64
EW = E // NW
NB = 160
EWP = NB * K
BLK = 16
NBLK = NB // BLK
R = 5
GD = 4
NPAD = 10112
RPT = NPAD // NS

_mesh = plsc.VectorSubcoreMesh(core_axis_name="c", subcore_axis_name="s")


def _deg_body(dst_hbm, out_hbm, dst_v, hist_v):
    c = lax.axis_index("c")
    s = lax.axis_index("s")
    wid = s * NC + c
    pltpu.sync_copy(dst_hbm.at[wid], dst_v)

    def _zero(i, carry):
        hist_v[pl.ds(i * 16, 16)] = jnp.zeros((16,), jnp.float32)
        return carry

    lax.fori_loop(0, NPAD // 16, _zero, 0, unroll=4)

    ones = jnp.ones((16,), jnp.float32)

    def _hist(i, carry):
        j = i // (K // 16)
        t = i % (K // 16)
        idx = dst_v[j, pl.ds(t * 16, 16)]
        plsc.addupdate_scatter(hist_v, [idx], ones)
        return carry

    lax.fori_loop(0, NB * (K // 16), _hist, 0, unroll=4)
    pltpu.sync_copy(hist_v, out_hbm.at[wid])


_deg_hist = functools.partial(
    pl.kernel,
    out_type=jax.ShapeDtypeStruct((NW, NPAD), jnp.float32),
    mesh=_mesh,
    scratch_types=[
        pltpu.VMEM((NB, K), jnp.int32),
        pltpu.VMEM((NPAD,), jnp.float32),
    ],
    compiler_params=pltpu.CompilerParams(needs_layout_passes=False),
)(_deg_body)


def _agg_body(g_hbm, src_hbm, dst_hbm, z_hbm, out_hbm, src_v, dst_v, rows_v,
              acc_sh, gsem, ssem, isem):
    c = lax.axis_index("c")
    s = lax.axis_index("s")
    wid = s * NC + c
    pltpu.sync_copy(z_hbm.at[pl.ds(s * RPT, RPT)],
                    acc_sh.at[pl.ds(s * RPT, RPT)])
    pltpu.sync_copy(src_hbm.at[wid, pl.ds(0, BLK)], src_v.at[0])
    pltpu.sync_copy(dst_hbm.at[wid, pl.ds(0, BLK)], dst_v.at[0])
    plsc.subcore_barrier()

    for q in range(GD):
        pltpu.async_copy(g_hbm.at[src_v.at[0, q]], rows_v.at[q], gsem.at[q])

    def _blk(b, carry):
        pb = lax.rem(b, 2)

        @pl.when(b + 1 < NBLK)
        def _prefetch_idx():
            pltpu.async_copy(src_hbm.at[wid, pl.ds((b + 1) * BLK, BLK)],
                             src_v.at[1 - pb], isem)
            pltpu.async_copy(dst_hbm.at[wid, pl.ds((b + 1) * BLK, BLK)],
                             dst_v.at[1 - pb], isem)

        def _edge(t, carry2):
            j = b * BLK + t
            p = lax.rem(j, R)
            tt = t + GD

            @pl.when(j >= 1)
            def _drain_prev_scatter():
                pltpu.make_async_copy(rows_v.at[lax.rem(j - 1, R)],
                                      acc_sh.at[dst_v.at[pb, t]],
                                      ssem.at[lax.rem(j - 1, R)]).wait()

            @pl.when(jnp.logical_and(tt == BLK, b + 1 < NBLK))
            def _wait_idx():
                pltpu.make_async_copy(
                    src_hbm.at[wid, pl.ds((b + 1) * BLK, BLK)],
                    src_v.at[1 - pb], isem).wait()
                pltpu.make_async_copy(
                    dst_hbm.at[wid, pl.ds((b + 1) * BLK, BLK)],
                    dst_v.at[1 - pb], isem).wait()

            pq = lax.rem(j + GD, R)

            @pl.when(tt < BLK)
            def _prefetch_same_blk():
                pltpu.async_copy(g_hbm.at[src_v.at[pb, tt]],
                                 rows_v.at[pq], gsem.at[pq])

            @pl.when(jnp.logical_and(tt >= BLK, b + 1 < NBLK))
            def _prefetch_next_blk():
                pltpu.async_copy(g_hbm.at[src_v.at[1 - pb, tt - BLK]],
                                 rows_v.at[pq], gsem.at[pq])

            pltpu.make_async_copy(g_hbm.at[src_v.at[pb, t]], rows_v.at[p],
                                  gsem.at[p]).wait()
            pltpu.async_copy(rows_v.at[p], acc_sh.at[dst_v.at[pb, t]],
                             ssem.at[p], add=True)

            return carry2

        lax.fori_loop(0, BLK, _edge, carry)
        return carry

    lax.fori_loop(0, NBLK, _blk, 0)
    pltpu.make_async_copy(rows_v.at[(NB - 1) % R],
                          acc_sh.at[dst_v.at[(NBLK - 1) % 2, BLK - 1]],
                          ssem.at[(NB - 1) % R]).wait()
    plsc.subcore_barrier()
    pltpu.sync_copy(acc_sh.at[pl.ds(s * RPT, RPT)],
                    out_hbm.at[c, pl.ds(s * RPT, RPT)])


_edge_agg = functools.partial(
    pl.kernel,
    out_type=jax.ShapeDtypeStruct((NC, NPAD, D), jnp.float32),
    mesh=_mesh,
    scratch_types=[
        pltpu.VMEM((2, BLK, K), jnp.int32),
        pltpu.VMEM((2, BLK, K), jnp.int32),
        pltpu.VMEM((R, K, D), jnp.float32),
        pltpu.VMEM_SHARED((NPAD, D), jnp.float32),
        pltpu.SemaphoreType.DMA((R,)),
        pltpu.SemaphoreType.DMA((R,)),
        pltpu.SemaphoreType.DMA,
    ],
)(_agg_body)


def _tc1_body(hist_ref, x_ref, w_ref, g_ref, dis_ref):
    deg = jnp.sum(hist_ref[...], axis=0) + 1.0
    dis = lax.rsqrt(deg)[:, None]
    h = jnp.dot(x_ref[...], w_ref[...], preferred_element_type=jnp.float32)
    g_ref[...] = h * dis[:N]
    dis_ref[...] = dis


_tc1 = pl.pallas_call(
    _tc1_body,
    out_shape=[
        jax.ShapeDtypeStruct((N, D), jnp.float32),
        jax.ShapeDtypeStruct((NPAD, 1), jnp.float32),
    ],
)


def _tc2_body(acc_ref, g_ref, dis_ref, w_ref, gamma_ref, beta_ref, g2_ref):
    dis = dis_ref[...][:N]
    pre = (acc_ref[0, :N, :] + acc_ref[1, :N, :] + g_ref[...]) * dis
    mu = jnp.mean(pre, axis=0)
    var = jnp.mean((pre - mu) ** 2, axis=0)
    y = (pre - mu) * lax.rsqrt(var + EPS) * gamma_ref[...] + beta_ref[...]
    y = jnp.maximum(y, 0.0)
    g2_ref[...] = jnp.dot(y, w_ref[...],
                          preferred_element_type=jnp.float32) * dis


_tc2 = pl.pallas_call(
    _tc2_body,
    out_shape=jax.ShapeDtypeStruct((N, D), jnp.float32),
)


def _tc3_body(acc_ref, g_ref, dis_ref, gamma_ref, beta_ref, out_ref):
    dis = dis_ref[...][:N]
    pre = (acc_ref[0, :N, :] + acc_ref[1, :N, :] + g_ref[...]) * dis
    mu = jnp.mean(pre, axis=0)
    var = jnp.mean((pre - mu) ** 2, axis=0)
    out_ref[...] = (pre - mu) * lax.rsqrt(var + EPS) * gamma_ref[...] \
        + beta_ref[...]


_tc3 = pl.pallas_call(
    _tc3_body,
    out_shape=jax.ShapeDtypeStruct((N, D), jnp.float32),
)


def kernel(x, edge_index, W1, b1, gamma1, beta1, W2, b2, gamma2, beta2):
    src = edge_index[0]
    dst = edge_index[1]
    npad = EWP - EW
    pad_lane = (jnp.arange(npad, dtype=jnp.int32) % (NPAD - N))
    src_p = jnp.concatenate(
        [src.reshape(NW, EW), jnp.broadcast_to(pad_lane, (NW, npad))],
        axis=1).reshape(NW, NB, K)
    dst_p = jnp.concatenate(
        [dst.reshape(NW, EW), jnp.broadcast_to(N + pad_lane, (NW, npad))],
        axis=1).reshape(NW, NB, K)
    zeros = jnp.zeros((NPAD, D), jnp.float32)

    hist = _deg_hist(dst_p)
    g1, dis = _tc1(hist, x, W1)
    acc1 = _edge_agg(g1, src_p, dst_p, zeros)
    g2 = _tc2(acc1, g1, dis, W2, gamma1, beta1)
    acc2 = _edge_agg(g2, src_p, dst_p, zeros)
    out = _tc3(acc2, g2, dis, gamma2, beta2)
    return out


# speedup vs baseline: 37.3634x; 1.0215x over previous
import functools

import jax
import jax.numpy as jnp
from jax import lax
from jax.experimental import pallas as pl
from jax.experimental.pallas import tpu as pltpu
from jax.experimental.pallas import tpu_sc as plsc

N = 10000
D = 128
E = 320000
EPS = 1e-5

NC = 2
NS = 16
NW = NC * NS
K = 64
EW = E // NW
NB = 160
EWP = NB * K
BLK = 16
NBLK = NB // BLK
R = 5
GD = 4
NPAD = 10112
RPT = NPAD // NS

_mesh = plsc.VectorSubcoreMesh(core_axis_name="c", subcore_axis_name="s")


def _deg_body(dst_hbm, out_hbm, dst_v, hist_v):
    c = lax.axis_index("c")
    s = lax.axis_index("s")
    wid = s * NC + c
    pltpu.sync_copy(dst_hbm.at[wid], dst_v)

    def _zero(i, carry):
        hist_v[pl.ds(i * 16, 16)] = jnp.zeros((16,), jnp.float32)
        return carry

    lax.fori_loop(0, NPAD // 16, _zero, 0, unroll=4)

    ones = jnp.ones((16,), jnp.float32)

    def _hist(i, carry):
        j = i // (K // 16)
        t = i % (K // 16)
        idx = dst_v[j, pl.ds(t * 16, 16)]
        plsc.addupdate_scatter(hist_v, [idx], ones)
        return carry

    lax.fori_loop(0, NB * (K // 16), _hist, 0, unroll=4)
    pltpu.sync_copy(hist_v, out_hbm.at[wid])


_deg_hist = functools.partial(
    pl.kernel,
    out_type=jax.ShapeDtypeStruct((NW, NPAD), jnp.float32),
    mesh=_mesh,
    scratch_types=[
        pltpu.VMEM((NB, K), jnp.int32),
        pltpu.VMEM((NPAD,), jnp.float32),
    ],
    compiler_params=pltpu.CompilerParams(needs_layout_passes=False),
)(_deg_body)


def _agg_body(g_hbm, src_hbm, dst_hbm, z_hbm, out_hbm, src_v, dst_v, rows_v,
              acc_sh, gsem, ssem, isem, zsem):
    c = lax.axis_index("c")
    s = lax.axis_index("s")
    wid = s * NC + c
    pltpu.async_copy(z_hbm.at[pl.ds(s * RPT, RPT)],
                     acc_sh.at[pl.ds(s * RPT, RPT)], zsem)
    pltpu.sync_copy(src_hbm.at[wid, pl.ds(0, BLK)], src_v.at[0])
    pltpu.sync_copy(dst_hbm.at[wid, pl.ds(0, BLK)], dst_v.at[0])

    for q in range(GD):
        pltpu.async_copy(g_hbm.at[src_v.at[0, q]], rows_v.at[q], gsem.at[q])
    pltpu.make_async_copy(z_hbm.at[pl.ds(s * RPT, RPT)],
                          acc_sh.at[pl.ds(s * RPT, RPT)], zsem).wait()
    plsc.subcore_barrier()

    def _blk(b, carry):
        pb = lax.rem(b, 2)

        @pl.when(b + 1 < NBLK)
        def _prefetch_idx():
            pltpu.async_copy(src_hbm.at[wid, pl.ds((b + 1) * BLK, BLK)],
                             src_v.at[1 - pb], isem)
            pltpu.async_copy(dst_hbm.at[wid, pl.ds((b + 1) * BLK, BLK)],
                             dst_v.at[1 - pb], isem)

        def _edge(t, carry2):
            j = b * BLK + t
            p = lax.rem(j, R)
            tt = t + GD

            @pl.when(j >= 1)
            def _drain_prev_scatter():
                pltpu.make_async_copy(rows_v.at[lax.rem(j - 1, R)],
                                      acc_sh.at[dst_v.at[pb, t]],
                                      ssem.at[lax.rem(j - 1, R)]).wait()

            @pl.when(jnp.logical_and(tt == BLK, b + 1 < NBLK))
            def _wait_idx():
                pltpu.make_async_copy(
                    src_hbm.at[wid, pl.ds((b + 1) * BLK, BLK)],
                    src_v.at[1 - pb], isem).wait()
                pltpu.make_async_copy(
                    dst_hbm.at[wid, pl.ds((b + 1) * BLK, BLK)],
                    dst_v.at[1 - pb], isem).wait()

            pq = lax.rem(j + GD, R)

            @pl.when(tt < BLK)
            def _prefetch_same_blk():
                pltpu.async_copy(g_hbm.at[src_v.at[pb, tt]],
                                 rows_v.at[pq], gsem.at[pq])

            @pl.when(jnp.logical_and(tt >= BLK, b + 1 < NBLK))
            def _prefetch_next_blk():
                pltpu.async_copy(g_hbm.at[src_v.at[1 - pb, tt - BLK]],
                                 rows_v.at[pq], gsem.at[pq])

            pltpu.make_async_copy(g_hbm.at[src_v.at[pb, t]], rows_v.at[p],
                                  gsem.at[p]).wait()
            pltpu.async_copy(rows_v.at[p], acc_sh.at[dst_v.at[pb, t]],
                             ssem.at[p], add=True)

            return carry2

        lax.fori_loop(0, BLK, _edge, carry)
        return carry

    lax.fori_loop(0, NBLK, _blk, 0)
    pltpu.make_async_copy(rows_v.at[(NB - 1) % R],
                          acc_sh.at[dst_v.at[(NBLK - 1) % 2, BLK - 1]],
                          ssem.at[(NB - 1) % R]).wait()
    plsc.subcore_barrier()
    pltpu.sync_copy(acc_sh.at[pl.ds(s * RPT, RPT)],
                    out_hbm.at[c, pl.ds(s * RPT, RPT)])


_edge_agg = functools.partial(
    pl.kernel,
    out_type=jax.ShapeDtypeStruct((NC, NPAD, D), jnp.float32),
    mesh=_mesh,
    scratch_types=[
        pltpu.VMEM((2, BLK, K), jnp.int32),
        pltpu.VMEM((2, BLK, K), jnp.int32),
        pltpu.VMEM((R, K, D), jnp.float32),
        pltpu.VMEM_SHARED((NPAD, D), jnp.float32),
        pltpu.SemaphoreType.DMA((R,)),
        pltpu.SemaphoreType.DMA((R,)),
        pltpu.SemaphoreType.DMA,
        pltpu.SemaphoreType.DMA,
    ],
)(_agg_body)


def _tc1_body(hist_ref, x_ref, w_ref, g_ref, dis_ref):
    deg = jnp.sum(hist_ref[...], axis=0) + 1.0
    dis = lax.rsqrt(deg)[:, None]
    h = jnp.dot(x_ref[...], w_ref[...], preferred_element_type=jnp.float32)
    g_ref[...] = h * dis[:N]
    dis_ref[...] = dis


_tc1 = pl.pallas_call(
    _tc1_body,
    out_shape=[
        jax.ShapeDtypeStruct((N, D), jnp.float32),
        jax.ShapeDtypeStruct((NPAD, 1), jnp.float32),
    ],
)


def _tc2_body(acc_ref, g_ref, dis_ref, w_ref, gamma_ref, beta_ref, g2_ref):
    dis = dis_ref[...][:N]
    pre = (acc_ref[0, :N, :] + acc_ref[1, :N, :] + g_ref[...]) * dis
    mu = jnp.mean(pre, axis=0)
    var = jnp.mean((pre - mu) ** 2, axis=0)
    y = (pre - mu) * lax.rsqrt(var + EPS) * gamma_ref[...] + beta_ref[...]
    y = jnp.maximum(y, 0.0)
    g2_ref[...] = jnp.dot(y, w_ref[...],
                          preferred_element_type=jnp.float32) * dis


_tc2 = pl.pallas_call(
    _tc2_body,
    out_shape=jax.ShapeDtypeStruct((N, D), jnp.float32),
)


def _tc3_body(acc_ref, g_ref, dis_ref, gamma_ref, beta_ref, out_ref):
    dis = dis_ref[...][:N]
    pre = (acc_ref[0, :N, :] + acc_ref[1, :N, :] + g_ref[...]) * dis
    mu = jnp.mean(pre, axis=0)
    var = jnp.mean((pre - mu) ** 2, axis=0)
    out_ref[...] = (pre - mu) * lax.rsqrt(var + EPS) * gamma_ref[...] \
        + beta_ref[...]


_tc3 = pl.pallas_call(
    _tc3_body,
    out_shape=jax.ShapeDtypeStruct((N, D), jnp.float32),
)


def kernel(x, edge_index, W1, b1, gamma1, beta1, W2, b2, gamma2, beta2):
    src = edge_index[0]
    dst = edge_index[1]
    npad = EWP - EW
    pad_lane = (jnp.arange(npad, dtype=jnp.int32) % (NPAD - N))
    src_p = jnp.concatenate(
        [src.reshape(NW, EW), jnp.broadcast_to(pad_lane, (NW, npad))],
        axis=1).reshape(NW, NB, K)
    dst_p = jnp.concatenate(
        [dst.reshape(NW, EW), jnp.broadcast_to(N + pad_lane, (NW, npad))],
        axis=1).reshape(NW, NB, K)
    zeros = jnp.zeros((NPAD, D), jnp.float32)

    hist = _deg_hist(dst_p)
    g1, dis = _tc1(hist, x, W1)
    acc1 = _edge_agg(g1, src_p, dst_p, zeros)
    g2 = _tc2(acc1, g1, dis, W2, gamma1, beta1)
    acc2 = _edge_agg(g2, src_p, dst_p, zeros)
    out = _tc3(acc2, g2, dis, gamma2, beta2)
    return out
